# Initial kernel scaffold; baseline (speedup 1.0000x reference)
import jax
import jax.numpy as jnp
from jax.experimental import pallas as pl


def kernel(out0, out1, out2, out3, comp_id):
    raise NotImplementedError("write your pallas kernel here")


# speedup vs baseline: 2.5249x; 2.5249x over previous
import jax
import jax.numpy as jnp
from jax.experimental import pallas as pl


def _select_body(c_ref, o0_ref, o1_ref, o2_ref, o3_ref, out_ref):
    cid = c_ref[...]
    s01 = jnp.where(cid == 0, o0_ref[...], o1_ref[...])
    s23 = jnp.where(cid == 2, o2_ref[...], o3_ref[...])
    out_ref[...] = jnp.where(cid < 2, s01, s23)


def kernel(out0, out1, out2, out3, comp_id):
    B, D = out0.shape
    BLK = 2048
    data_spec = pl.BlockSpec((BLK, D), lambda i: (i, 0))
    return pl.pallas_call(
        _select_body,
        grid=(B // BLK,),
        in_specs=[
            pl.BlockSpec((BLK, 1), lambda i: (i, 0)),
            data_spec, data_spec, data_spec, data_spec,
        ],
        out_specs=data_spec,
        out_shape=jax.ShapeDtypeStruct((B, D), jnp.float32),
    )(comp_id, out0, out1, out2, out3)
